# baseline (speedup 1.0000x reference)
import functools

import jax
import jax.numpy as jnp
from jax import lax
from jax.experimental import pallas as pl
from jax.experimental.pallas import tpu as pltpu
from jax.experimental.pallas import tpu_sc as plsc

N = 10000
H = 16
NPAD = 10240
NC = 2
NS = 16
NW = NC * NS
CH = 128
RPT = NPAD // NS
SUPE = 1024
NBUF = 4
LOOK = 2

f32 = jnp.float32


FAST_CID = 1
FAST_FRAC_NUM = 5


@functools.lru_cache(maxsize=None)
def _sc_aggregate(with_cnt, spt_fast, spt_slow):
  mesh = plsc.VectorSubcoreMesh(core_axis_name="c", subcore_axis_name="s")

  def body(src_h, dst_h, tab_h, z_h, ones_h, *rest):
    it = iter(rest)
    agg_out = next(it)
    cnt_out = next(it) if with_cnt else None
    sidx = next(it)
    didx = next(it)
    rows = [next(it) for _ in range(NBUF)]
    ones_v = next(it) if with_cnt else None
    agg_s = next(it)
    cnt_s = next(it) if with_cnt else None
    gs = [next(it) for _ in range(NBUF)]
    ss = [next(it) for _ in range(NBUF)]
    cs = next(it) if with_cnt else None
    cid = lax.axis_index("c")
    sid = lax.axis_index("s")
    t0 = sid * RPT
    on_fast = cid == FAST_CID
    spt = lax.select(on_fast, spt_fast, spt_slow)
    base = lax.select(on_fast, sid * spt_fast,
                      NS * spt_fast + sid * spt_slow)

    pltpu.sync_copy(z_h, agg_s.at[pl.ds(t0, RPT)])
    if with_cnt:
      pltpu.sync_copy(z_h, cnt_s.at[pl.ds(t0, RPT)])
      pltpu.sync_copy(ones_h, ones_v)
    pltpu.sync_copy(src_h.at[pl.ds(base, spt_fast)], sidx)
    pltpu.sync_copy(dst_h.at[pl.ds(base, spt_fast)], didx)
    plsc.subcore_barrier()

    def gather(j, b):
      return pltpu.make_async_copy(tab_h.at[sidx.at[j]], rows[b], gs[b])

    def scatter(j, b):
      return pltpu.make_async_copy(rows[b], agg_s.at[didx.at[j]], ss[b])

    def cscatter(j):
      return pltpu.make_async_copy(ones_v, cnt_s.at[didx.at[j]], cs)

    for b in range(LOOK):
      gather(b, b).start()

    def round_(r, carry):
      for i in range(NBUF):
        j = r * NBUF + i
        jf = j + LOOK
        bf = (i + LOOK) % NBUF

        @pl.when(jf < spt)
        def _(jf=jf, bf=bf):
          @pl.when(jf >= NBUF)
          def _():
            scatter(jf - NBUF, bf).wait()
          gather(jf, bf).start()

        @pl.when(j < spt)
        def _(j=j, i=i):
          gather(j, i).wait()
          scatter(j, i).start(add=True)
          if with_cnt:
            cscatter(j).start(add=True)

            @pl.when(j >= NBUF)
            def _():
              cscatter(j).wait()
      return carry

    lax.fori_loop(0, (spt + NBUF - 1) // NBUF, round_, 0)
    for i in range(NBUF):
      scatter(0, i).wait()
      if with_cnt:
        cscatter(0).wait()
    plsc.subcore_barrier()

    pltpu.sync_copy(agg_s.at[pl.ds(t0, RPT)],
                    agg_out.at[pl.ds(cid * NPAD + t0, RPT)])
    if with_cnt:
      pltpu.sync_copy(cnt_s.at[pl.ds(t0, RPT)],
                      cnt_out.at[pl.ds(cid * NPAD + t0, RPT)])

  out_type = [jax.ShapeDtypeStruct((NC * NPAD, H), f32)]
  if with_cnt:
    out_type.append(jax.ShapeDtypeStruct((NC * NPAD, H), f32))
  scratch = [pltpu.VMEM((spt_fast, SUPE), jnp.int32),
             pltpu.VMEM((spt_fast, SUPE), jnp.int32)]
  scratch.extend([pltpu.VMEM((SUPE, H), f32)] * NBUF)
  if with_cnt:
    scratch.append(pltpu.VMEM((SUPE, H), f32))
  scratch.append(pltpu.VMEM_SHARED((NPAD, H), f32))
  if with_cnt:
    scratch.append(pltpu.VMEM_SHARED((NPAD, H), f32))
  scratch.extend([pltpu.SemaphoreType.DMA] * (2 * NBUF + (1 if with_cnt
                                                          else 0)))

  return pl.kernel(
      body, out_type=tuple(out_type), mesh=mesh,
      scratch_types=tuple(scratch),
      compiler_params=pltpu.CompilerParams(use_tc_tiling_on_sc=False))


def _tc_pre(x, Wlt, Wrt):
  def body(x_ref, wl_ref, wr_ref, p_ref, r_ref):
    xv = x_ref[...]
    p_ref[...] = jnp.dot(xv, wl_ref[...], preferred_element_type=f32)
    r_ref[...] = jnp.dot(xv, wr_ref[...], preferred_element_type=f32)
  return pl.pallas_call(
      body,
      out_shape=(jax.ShapeDtypeStruct((N, H), f32),
                 jax.ShapeDtypeStruct((N, H), f32)),
  )(x, Wlt, Wrt)


def _tc_mid(agg, cnt, r1, b1, Wlt, Wrt):
  def body(a_ref, c_ref, r_ref, b_ref, wl_ref, wr_ref, p_ref, q_ref):
    asum = a_ref[0:NPAD, :] + a_ref[NPAD:2 * NPAD, :]
    csum = c_ref[0:NPAD, :] + c_ref[NPAD:2 * NPAD, :]
    mean = (asum / jnp.maximum(csum, 1.0))[:N]
    h = jax.nn.sigmoid(mean + b_ref[...] + r_ref[...])
    p_ref[...] = jnp.dot(h, wl_ref[...], preferred_element_type=f32)
    q_ref[...] = jnp.dot(h, wr_ref[...], preferred_element_type=f32)
  return pl.pallas_call(
      body,
      out_shape=(jax.ShapeDtypeStruct((N, H), f32),
                 jax.ShapeDtypeStruct((N, H), f32)),
  )(agg, cnt, r1, b1, Wlt, Wrt)


def _tc_post(agg, cnt, r2, b2, W1t, bl1, W2t, bl2):
  def body(a_ref, c_ref, r_ref, b_ref, w1_ref, b1_ref, w2_ref, b2_ref,
           o_ref):
    asum = a_ref[0:NPAD, :] + a_ref[NPAD:2 * NPAD, :]
    csum = c_ref[0:NPAD, :] + c_ref[NPAD:2 * NPAD, :]
    mean = (asum / jnp.maximum(csum, 1.0))[:N]
    h2 = jax.nn.sigmoid(mean + b_ref[...] + r_ref[...])
    h3 = jax.nn.sigmoid(
        jnp.dot(h2, w1_ref[...], preferred_element_type=f32) + b1_ref[...])
    o_ref[...] = (jnp.dot(h3, w2_ref[...], preferred_element_type=f32)
                  + b2_ref[...])
  return pl.pallas_call(
      body,
      out_shape=jax.ShapeDtypeStruct((N, 16), f32),
  )(agg, cnt, r2, b2, W1t, bl1, W2t, bl2)


def kernel(x, edge_list, W1l, b1, W1r, W2l, b2, W2r, Wlin1, blin1, Wlin2,
           blin2):
  el = edge_list.astype(jnp.int32)
  src, dst = el[0], el[1]
  e = src.shape[0]
  align = NW * SUPE
  epad = -(-e // align) * align
  tpt = epad // SUPE // NS
  spt_fast = round(tpt * FAST_FRAC_NUM / 10)
  spt_slow = tpt - spt_fast
  epad2 = epad + (spt_fast - spt_slow) * SUPE
  src_p = jnp.concatenate(
      [src, jnp.zeros((epad2 - e,), jnp.int32)]).reshape(-1, SUPE)
  dst_p = jnp.concatenate(
      [dst, jnp.full((epad2 - e,), NPAD - 1, jnp.int32)]).reshape(-1, SUPE)
  zrows = jnp.zeros((RPT, H), f32)
  ones = jnp.ones((SUPE, H), f32)

  p1, r1 = _tc_pre(x, W1l.T, W1r.T)
  agg1, cnt = _sc_aggregate(True, spt_fast, spt_slow)(
      src_p, dst_p, p1, zrows, ones)
  p2, r2 = _tc_mid(agg1, cnt, r1, b1.reshape(1, H), W2l.T, W2r.T)
  (agg2,) = _sc_aggregate(False, spt_fast, spt_slow)(
      src_p, dst_p, p2, zrows, ones)
  return _tc_post(agg2, cnt, r2, b2.reshape(1, H), Wlin1.T,
                  blin1.reshape(1, H), Wlin2.T, blin2.reshape(1, 16))


# speedup vs baseline: 1.1621x; 1.1621x over previous
import functools

import jax
import jax.numpy as jnp
from jax import lax
from jax.experimental import pallas as pl
from jax.experimental.pallas import tpu as pltpu
from jax.experimental.pallas import tpu_sc as plsc

N = 10000
H = 16
NPAD = 10240
NC = 2
NS = 16
NW = NC * NS
CH = 128
RPT = NPAD // NS
SUPE = 1024
NBUF = 4
LOOK = 2

f32 = jnp.float32


FAST_CID = 1
FAST_FRAC_NUM = 7


@functools.lru_cache(maxsize=None)
def _sc_aggregate(with_cnt, spt_fast, spt_slow):
  mesh = plsc.VectorSubcoreMesh(core_axis_name="c", subcore_axis_name="s")

  def body(src_h, dst_h, tab_h, z_h, ones_h, *rest):
    it = iter(rest)
    agg_out = next(it)
    cnt_out = next(it) if with_cnt else None
    sidx = next(it)
    didx = next(it)
    rows = [next(it) for _ in range(NBUF)]
    ones_v = next(it) if with_cnt else None
    tab_s = None if with_cnt else next(it)
    agg_s = next(it)
    cnt_s = next(it) if with_cnt else None
    gs = [next(it) for _ in range(NBUF)]
    ss = [next(it) for _ in range(NBUF)]
    cs = next(it) if with_cnt else None
    cid = lax.axis_index("c")
    sid = lax.axis_index("s")
    t0 = sid * RPT
    on_fast = cid == FAST_CID
    spt = lax.select(on_fast, spt_fast, spt_slow)
    base = lax.select(on_fast, sid * spt_fast,
                      NS * spt_fast + sid * spt_slow)

    pltpu.sync_copy(z_h, agg_s.at[pl.ds(t0, RPT)])
    if with_cnt:
      pltpu.sync_copy(z_h, cnt_s.at[pl.ds(t0, RPT)])
      pltpu.sync_copy(ones_h, ones_v)
    pltpu.sync_copy(src_h.at[pl.ds(base, spt_fast)], sidx)
    pltpu.sync_copy(dst_h.at[pl.ds(base, spt_fast)], didx)
    if tab_s is not None:
      pltpu.sync_copy(tab_h.at[pl.ds(t0, RPT)], tab_s.at[pl.ds(t0, RPT)])
    plsc.subcore_barrier()
    tab = tab_h if tab_s is None else tab_s

    def gather(j, b):
      return pltpu.make_async_copy(tab.at[sidx.at[j]], rows[b], gs[b])

    def scatter(j, b):
      return pltpu.make_async_copy(rows[b], agg_s.at[didx.at[j]], ss[b])

    def cscatter(j):
      return pltpu.make_async_copy(ones_v, cnt_s.at[didx.at[j]], cs)

    for b in range(LOOK):
      gather(b, b).start()

    def round_(r, carry):
      for i in range(NBUF):
        j = r * NBUF + i
        jf = j + LOOK
        bf = (i + LOOK) % NBUF

        @pl.when(jf < spt)
        def _(jf=jf, bf=bf):
          @pl.when(jf >= NBUF)
          def _():
            scatter(jf - NBUF, bf).wait()
          gather(jf, bf).start()

        @pl.when(j < spt)
        def _(j=j, i=i):
          gather(j, i).wait()
          scatter(j, i).start(add=True)
          if with_cnt:
            cscatter(j).start(add=True)

            @pl.when(j >= NBUF)
            def _():
              cscatter(j).wait()
      return carry

    lax.fori_loop(0, (spt + NBUF - 1) // NBUF, round_, 0)
    for i in range(NBUF):
      scatter(0, i).wait()
      if with_cnt:
        cscatter(0).wait()
    plsc.subcore_barrier()

    pltpu.sync_copy(agg_s.at[pl.ds(t0, RPT)],
                    agg_out.at[pl.ds(cid * NPAD + t0, RPT)])
    if with_cnt:
      pltpu.sync_copy(cnt_s.at[pl.ds(t0, RPT)],
                      cnt_out.at[pl.ds(cid * NPAD + t0, RPT)])

  out_type = [pltpu.HBM((NC * NPAD, H), f32)]
  if with_cnt:
    out_type.append(pltpu.HBM((NC * NPAD, H), f32))
  scratch = [pltpu.VMEM((spt_fast, SUPE), jnp.int32),
             pltpu.VMEM((spt_fast, SUPE), jnp.int32)]
  scratch.extend([pltpu.VMEM((SUPE, H), f32)] * NBUF)
  if with_cnt:
    scratch.append(pltpu.VMEM((SUPE, H), f32))
  else:
    scratch.append(pltpu.VMEM_SHARED((NPAD, H), f32))
  scratch.append(pltpu.VMEM_SHARED((NPAD, H), f32))
  if with_cnt:
    scratch.append(pltpu.VMEM_SHARED((NPAD, H), f32))
  scratch.extend([pltpu.SemaphoreType.DMA] * (2 * NBUF + (1 if with_cnt
                                                          else 0)))

  return pl.kernel(
      body, out_type=tuple(out_type), mesh=mesh,
      scratch_types=tuple(scratch),
      compiler_params=pltpu.CompilerParams(use_tc_tiling_on_sc=False))


def _tc_pre(x, Wlt, Wrt):
  def body(x_ref, wl_ref, wr_ref, p_ref, r_ref):
    xv = x_ref[...]
    p_ref[0:N, :] = jnp.dot(xv, wl_ref[...], preferred_element_type=f32)
    p_ref[N:NPAD, :] = jnp.zeros((NPAD - N, H), f32)
    r_ref[...] = jnp.dot(xv, wr_ref[...], preferred_element_type=f32)
  return pl.pallas_call(
      body,
      out_shape=(jax.ShapeDtypeStruct((NPAD, H), f32),
                 jax.ShapeDtypeStruct((N, H), f32)),
  )(x, Wlt, Wrt)


def _tc_mid(agg, cnt, r1, b1, Wlt, Wrt):
  def body(a_ref, c_ref, r_ref, b_ref, wl_ref, wr_ref, p_ref, q_ref):
    asum = a_ref[0:NPAD, :] + a_ref[NPAD:2 * NPAD, :]
    csum = c_ref[0:NPAD, :] + c_ref[NPAD:2 * NPAD, :]
    mean = (asum / jnp.maximum(csum, 1.0))[:N]
    h = jax.nn.sigmoid(mean + b_ref[...] + r_ref[...])
    p_ref[0:N, :] = jnp.dot(h, wl_ref[...], preferred_element_type=f32)
    p_ref[N:NPAD, :] = jnp.zeros((NPAD - N, H), f32)
    q_ref[...] = jnp.dot(h, wr_ref[...], preferred_element_type=f32)
  return pl.pallas_call(
      body,
      out_shape=(jax.ShapeDtypeStruct((NPAD, H), f32),
                 jax.ShapeDtypeStruct((N, H), f32)),
  )(agg, cnt, r1, b1, Wlt, Wrt)


def _tc_post(agg, cnt, r2, b2, W1t, bl1, W2t, bl2):
  def body(a_ref, c_ref, r_ref, b_ref, w1_ref, b1_ref, w2_ref, b2_ref,
           o_ref):
    asum = a_ref[0:NPAD, :] + a_ref[NPAD:2 * NPAD, :]
    csum = c_ref[0:NPAD, :] + c_ref[NPAD:2 * NPAD, :]
    mean = (asum / jnp.maximum(csum, 1.0))[:N]
    h2 = jax.nn.sigmoid(mean + b_ref[...] + r_ref[...])
    h3 = jax.nn.sigmoid(
        jnp.dot(h2, w1_ref[...], preferred_element_type=f32) + b1_ref[...])
    o_ref[...] = (jnp.dot(h3, w2_ref[...], preferred_element_type=f32)
                  + b2_ref[...])
  return pl.pallas_call(
      body,
      out_shape=jax.ShapeDtypeStruct((N, 16), f32),
  )(agg, cnt, r2, b2, W1t, bl1, W2t, bl2)


def kernel(x, edge_list, W1l, b1, W1r, W2l, b2, W2r, Wlin1, blin1, Wlin2,
           blin2):
  el = edge_list.astype(jnp.int32)
  src, dst = el[0], el[1]
  e = src.shape[0]
  align = NW * SUPE
  epad = -(-e // align) * align
  tpt = epad // SUPE // NS
  spt_fast = round(tpt * FAST_FRAC_NUM / 10)
  spt_slow = tpt - spt_fast
  epad2 = epad + (spt_fast - spt_slow) * SUPE
  src_p = jnp.concatenate(
      [src, jnp.zeros((epad2 - e,), jnp.int32)]).reshape(-1, SUPE)
  dst_p = jnp.concatenate(
      [dst, jnp.full((epad2 - e,), NPAD - 1, jnp.int32)]).reshape(-1, SUPE)
  zrows = jnp.zeros((RPT, H), f32)
  ones = jnp.ones((SUPE, H), f32)

  hbm = lambda a: pltpu.with_memory_space_constraint(a, pltpu.HBM)
  src_p = hbm(src_p)
  dst_p = hbm(dst_p)
  p1, r1 = _tc_pre(x, W1l.T, W1r.T)
  agg1, cnt = _sc_aggregate(True, spt_fast, spt_slow)(
      src_p, dst_p, hbm(p1), hbm(zrows), hbm(ones))
  p2, r2 = _tc_mid(agg1, cnt, r1, b1.reshape(1, H), W2l.T, W2r.T)
  (agg2,) = _sc_aggregate(False, spt_fast, spt_slow)(
      src_p, dst_p, hbm(p2), hbm(zrows), hbm(ones))
  return _tc_post(agg2, cnt, r2, b2.reshape(1, H), Wlin1.T,
                  blin1.reshape(1, H), Wlin2.T, blin2.reshape(1, 16))


# speedup vs baseline: 1.4561x; 1.2530x over previous
import functools

import jax
import jax.numpy as jnp
from jax import lax
from jax.experimental import pallas as pl
from jax.experimental.pallas import tpu as pltpu
from jax.experimental.pallas import tpu_sc as plsc

N = 10000
H = 16
NPAD = 10240
NC = 2
NS = 16
NW = NC * NS
CH = 128
RPT = NPAD // NS
SUPE = 1024
NBUF = 4
LOOK = 2

f32 = jnp.float32


FAST_CID = 1
FAST_FRAC_NUM = 7


@functools.lru_cache(maxsize=None)
def _sc_aggregate(with_cnt, spt_fast, spt_slow):
  mesh = plsc.VectorSubcoreMesh(core_axis_name="c", subcore_axis_name="s")

  def body(src_h, dst_h, tab_h, z_h, z1_h, ones_h, *rest):
    it = iter(rest)
    agg_out = next(it)
    cnt_out = next(it) if with_cnt else None
    sidx = next(it)
    didx = next(it)
    rows = [next(it) for _ in range(NBUF)]
    ones_v = next(it) if with_cnt else None
    tab_s = next(it)
    agg_s = next(it)
    cnt_s = next(it) if with_cnt else None
    gs = [next(it) for _ in range(NBUF)]
    ss = [next(it) for _ in range(NBUF)]
    cs = next(it) if with_cnt else None
    cid = lax.axis_index("c")
    sid = lax.axis_index("s")
    t0 = sid * RPT
    on_fast = cid == FAST_CID
    spt = lax.select(on_fast, spt_fast, spt_slow)
    base = lax.select(on_fast, sid * spt_fast,
                      NS * spt_fast + sid * spt_slow)

    pltpu.sync_copy(z_h, agg_s.at[pl.ds(t0, RPT)])
    if with_cnt:
      pltpu.sync_copy(z1_h, cnt_s.at[pl.ds(t0, RPT)])
      pltpu.sync_copy(ones_h, ones_v)
    pltpu.sync_copy(src_h.at[pl.ds(base, spt_fast)], sidx)
    pltpu.sync_copy(dst_h.at[pl.ds(base, spt_fast)], didx)
    pltpu.sync_copy(tab_h.at[pl.ds(t0, RPT)], tab_s.at[pl.ds(t0, RPT)])
    plsc.subcore_barrier()

    def gather(j, b):
      return pltpu.make_async_copy(tab_s.at[sidx.at[j]], rows[b], gs[b])

    def scatter(j, b):
      return pltpu.make_async_copy(rows[b], agg_s.at[didx.at[j]], ss[b])

    def cscatter(j):
      return pltpu.make_async_copy(ones_v, cnt_s.at[didx.at[j]], cs)

    for b in range(LOOK):
      gather(b, b).start()

    def round_(r, carry):
      for i in range(NBUF):
        j = r * NBUF + i
        jf = j + LOOK
        bf = (i + LOOK) % NBUF

        @pl.when(jf < spt)
        def _(jf=jf, bf=bf):
          @pl.when(jf >= NBUF)
          def _():
            scatter(jf - NBUF, bf).wait()
          gather(jf, bf).start()

        @pl.when(j < spt)
        def _(j=j, i=i):
          gather(j, i).wait()
          scatter(j, i).start(add=True)
          if with_cnt:
            cscatter(j).start(add=True)

            @pl.when(j >= NBUF)
            def _():
              cscatter(j).wait()
      return carry

    lax.fori_loop(0, (spt + NBUF - 1) // NBUF, round_, 0)
    for i in range(NBUF):
      scatter(0, i).wait()
      if with_cnt:
        cscatter(0).wait()
    plsc.subcore_barrier()

    pltpu.sync_copy(agg_s.at[pl.ds(t0, RPT)],
                    agg_out.at[pl.ds(cid * NPAD + t0, RPT)])
    if with_cnt:
      pltpu.sync_copy(cnt_s.at[pl.ds(t0, RPT)],
                      cnt_out.at[pl.ds(cid * NPAD + t0, RPT)])
    del cnt_out

  out_type = [pltpu.HBM((NC * NPAD, H), f32)]
  if with_cnt:
    out_type.append(pltpu.HBM((NC * NPAD,), f32))
  scratch = [pltpu.VMEM((spt_fast, SUPE), jnp.int32),
             pltpu.VMEM((spt_fast, SUPE), jnp.int32)]
  scratch.extend([pltpu.VMEM((SUPE, H), f32)] * NBUF)
  if with_cnt:
    scratch.append(pltpu.VMEM((SUPE,), f32))
  scratch.append(pltpu.VMEM_SHARED((NPAD, H), f32))
  scratch.append(pltpu.VMEM_SHARED((NPAD, H), f32))
  if with_cnt:
    scratch.append(pltpu.VMEM_SHARED((NPAD,), f32))
  scratch.extend([pltpu.SemaphoreType.DMA] * (2 * NBUF + (1 if with_cnt
                                                          else 0)))

  return pl.kernel(
      body, out_type=tuple(out_type), mesh=mesh,
      scratch_types=tuple(scratch),
      compiler_params=pltpu.CompilerParams(use_tc_tiling_on_sc=False))


def _tc_pre(x, Wlt, Wrt):
  def body(x_ref, wl_ref, wr_ref, p_ref, r_ref):
    xv = x_ref[...]
    p_ref[0:N, :] = jnp.dot(xv, wl_ref[...], preferred_element_type=f32)
    p_ref[N:NPAD, :] = jnp.zeros((NPAD - N, H), f32)
    r_ref[...] = jnp.dot(xv, wr_ref[...], preferred_element_type=f32)
  return pl.pallas_call(
      body,
      out_shape=(jax.ShapeDtypeStruct((NPAD, H), f32),
                 jax.ShapeDtypeStruct((N, H), f32)),
  )(x, Wlt, Wrt)


def _tc_mid(agg, cnt, r1, b1, Wlt, Wrt):
  def body(a_ref, c_ref, r_ref, b_ref, wl_ref, wr_ref, p_ref, q_ref):
    asum = a_ref[0:NPAD, :] + a_ref[NPAD:2 * NPAD, :]
    csum = c_ref[0:NPAD, :] + c_ref[NPAD:2 * NPAD, :]
    mean = (asum / jnp.maximum(csum, 1.0))[:N]
    h = jax.nn.sigmoid(mean + b_ref[...] + r_ref[...])
    p_ref[0:N, :] = jnp.dot(h, wl_ref[...], preferred_element_type=f32)
    p_ref[N:NPAD, :] = jnp.zeros((NPAD - N, H), f32)
    q_ref[...] = jnp.dot(h, wr_ref[...], preferred_element_type=f32)
  return pl.pallas_call(
      body,
      out_shape=(jax.ShapeDtypeStruct((NPAD, H), f32),
                 jax.ShapeDtypeStruct((N, H), f32)),
  )(agg, cnt, r1, b1, Wlt, Wrt)


def _tc_post(agg, cnt, r2, b2, W1t, bl1, W2t, bl2):
  def body(a_ref, c_ref, r_ref, b_ref, w1_ref, b1_ref, w2_ref, b2_ref,
           o_ref):
    asum = a_ref[0:NPAD, :] + a_ref[NPAD:2 * NPAD, :]
    csum = c_ref[0:NPAD, :] + c_ref[NPAD:2 * NPAD, :]
    mean = (asum / jnp.maximum(csum, 1.0))[:N]
    h2 = jax.nn.sigmoid(mean + b_ref[...] + r_ref[...])
    h3 = jax.nn.sigmoid(
        jnp.dot(h2, w1_ref[...], preferred_element_type=f32) + b1_ref[...])
    o_ref[...] = (jnp.dot(h3, w2_ref[...], preferred_element_type=f32)
                  + b2_ref[...])
  return pl.pallas_call(
      body,
      out_shape=jax.ShapeDtypeStruct((N, 16), f32),
  )(agg, cnt, r2, b2, W1t, bl1, W2t, bl2)


def kernel(x, edge_list, W1l, b1, W1r, W2l, b2, W2r, Wlin1, blin1, Wlin2,
           blin2):
  el = edge_list.astype(jnp.int32)
  src, dst = el[0], el[1]
  e = src.shape[0]
  align = NW * SUPE
  epad = -(-e // align) * align
  tpt = epad // SUPE // NS
  spt_fast = round(tpt * FAST_FRAC_NUM / 10)
  spt_slow = tpt - spt_fast
  epad2 = epad + (spt_fast - spt_slow) * SUPE
  src_p = jnp.concatenate(
      [src, jnp.zeros((epad2 - e,), jnp.int32)]).reshape(-1, SUPE)
  dst_p = jnp.concatenate(
      [dst, jnp.full((epad2 - e,), NPAD - 1, jnp.int32)]).reshape(-1, SUPE)
  zrows = jnp.zeros((RPT, H), f32)
  zcnt = jnp.zeros((RPT,), f32)
  ones = jnp.ones((SUPE,), f32)

  hbm = lambda a: pltpu.with_memory_space_constraint(a, pltpu.HBM)
  src_p = hbm(src_p)
  dst_p = hbm(dst_p)
  p1, r1 = _tc_pre(x, W1l.T, W1r.T)
  agg1, cnt = _sc_aggregate(True, spt_fast, spt_slow)(
      src_p, dst_p, hbm(p1), hbm(zrows), hbm(zcnt), hbm(ones))
  cnt = cnt.reshape(NC * NPAD, 1)
  p2, r2 = _tc_mid(agg1, cnt, r1, b1.reshape(1, H), W2l.T, W2r.T)
  (agg2,) = _sc_aggregate(False, spt_fast, spt_slow)(
      src_p, dst_p, hbm(p2), hbm(zrows), hbm(zcnt), hbm(ones))
  return _tc_post(agg2, cnt, r2, b2.reshape(1, H), Wlin1.T,
                  blin1.reshape(1, H), Wlin2.T, blin2.reshape(1, 16))
